# baseline (speedup 1.0000x reference)
import functools

import jax
import jax.numpy as jnp
from jax import lax
from jax.experimental import pallas as pl
from jax.experimental.pallas import tpu as pltpu
from jax.experimental.pallas import tpu_sc as plsc

TOKENS = 8192
DM = 4096
NE = 64
TOPK = 8
NWORK = 32
CHUNK = TOKENS * TOPK // NWORK
BT = TOKENS // NWORK


def _gate_block(x_ref, w_ref, gate_ref, idx_ref, cumx_ref,
                tpe_ref, laux_ref, me_acc, hist_acc):
    i = pl.program_id(0)

    @pl.when(i == 0)
    def _init():
        me_acc[...] = jnp.zeros_like(me_acc)
        hist_acc[...] = jnp.zeros_like(hist_acc)

    cumx_ref[0, 0, :] = hist_acc[0, :].astype(jnp.int32)

    logits = lax.dot_general(x_ref[...], w_ref[...],
                             (((1,), (1,)), ((), ())),
                             preferred_element_type=jnp.float32)

    iota_f = lax.broadcasted_iota(jnp.int32, (BT, NE), 1).astype(jnp.float32)
    cur = logits
    row_max = None
    gates = []
    idxs = []
    for j in range(TOPK):
        m = jnp.max(cur, axis=1, keepdims=True)
        if j == 0:
            row_max = m
        idx = jnp.min(jnp.where(cur == m, iota_f, float(NE)),
                      axis=1, keepdims=True)
        gates.append(m)
        idxs.append(idx)
        cur = jnp.where(iota_f == idx, -jnp.inf, cur)

    gate_ref[...] = jnp.concatenate(gates, axis=1)
    idx_ref[...] = jnp.concatenate(idxs, axis=1).astype(jnp.int32)

    ex = jnp.exp(logits - row_max)
    scores = ex / jnp.sum(ex, axis=1, keepdims=True)
    me_acc[0, :] = me_acc[0, :] + jnp.sum(scores, axis=0)
    sel_acc = (cur == -jnp.inf).astype(jnp.float32)
    hist_acc[0, :] = hist_acc[0, :] + jnp.sum(sel_acc, axis=0)

    @pl.when(i == NWORK - 1)
    def _final():
        tpe_f = hist_acc[0, :]
        tpe_ref[0, :] = tpe_f.astype(jnp.int32)
        me = me_acc[0, :] * (1.0 / TOKENS)
        ce = tpe_f * (1.0 / TOKENS)
        laux_ref[...] = jnp.sum(me * ce).reshape(1, 1) * (NE / TOPK)


_gate_call = pl.pallas_call(
    _gate_block,
    grid=(NWORK,),
    in_specs=[
        pl.BlockSpec((BT, DM), lambda i: (i, 0)),
        pl.BlockSpec((NE, DM), lambda i: (0, 0)),
    ],
    out_specs=[
        pl.BlockSpec((BT, TOPK), lambda i: (i, 0)),
        pl.BlockSpec((BT, TOPK), lambda i: (i, 0)),
        pl.BlockSpec((1, 1, NE), lambda i: (i, 0, 0)),
        pl.BlockSpec((1, NE), lambda i: (0, 0)),
        pl.BlockSpec((1, 1), lambda i: (0, 0)),
    ],
    out_shape=[
        jax.ShapeDtypeStruct((TOKENS, TOPK), jnp.float32),
        jax.ShapeDtypeStruct((TOKENS, TOPK), jnp.int32),
        jax.ShapeDtypeStruct((NWORK, 1, NE), jnp.int32),
        jax.ShapeDtypeStruct((1, NE), jnp.int32),
        jax.ShapeDtypeStruct((1, 1), jnp.float32),
    ],
    scratch_shapes=[
        pltpu.VMEM((1, NE), jnp.float32),
        pltpu.VMEM((1, NE), jnp.float32),
    ],
)


def _sort_body(te_hbm, tpe_hbm, cumx_hbm, idxout_hbm, binout_hbm, bins_hbm,
               te_v, src_v, cnt_v, tmp_v, bin_v, *pos_rows_and_sem):
    pos_rows = pos_rows_and_sem[:16]
    sem = pos_rows_and_sem[16]
    c = lax.axis_index("c")
    s = lax.axis_index("s")
    wid = s * 2 + c

    pltpu.sync_copy(te_hbm.at[pl.ds(wid * 16, 16)], te_v)
    pltpu.sync_copy(tpe_hbm.at[0], tmp_v)
    pltpu.sync_copy(cumx_hbm.at[wid, 0], cnt_v)

    carry = jnp.int32(0)
    for t in range(4):
        sl = pl.ds(t * 16, 16)
        v = tmp_v[sl]
        incl = plsc.cumsum(v)
        cnt_v[sl] = cnt_v[sl] + (carry + incl - v)
        bin_v[sl] = carry + incl
        carry = carry + jnp.sum(v)

    @pl.when(wid == 0)
    def _bins_out():
        pltpu.sync_copy(bin_v, bins_hbm)

    base = wid * CHUNK
    for r in range(16):
        for q in range(8):
            sl = pl.ds(q * 16, 16)
            keys = te_v[r, sl]
            cnt, last = plsc.scan_count(keys)
            b = plsc.load_gather(cnt_v, [keys])
            pos_rows[r][sl] = b + cnt - 1
            plsc.store_scatter(cnt_v, [keys], b + cnt, mask=last)
            src_v[r, sl] = base + r * 128 + q * 16 + lax.iota(jnp.int32, 16)

    copies = []
    for r in range(16):
        copies.append(pltpu.async_copy(src_v.at[r], idxout_hbm.at[pos_rows[r]], sem))
        copies.append(pltpu.async_copy(te_v.at[r], binout_hbm.at[pos_rows[r]], sem))
    for d in copies:
        d.wait()


@functools.cache
def _sort_call():
    mesh = plsc.VectorSubcoreMesh(core_axis_name="c", subcore_axis_name="s",
                                  num_cores=2, num_subcores=16)
    return pl.kernel(
        _sort_body,
        out_type=[
            jax.ShapeDtypeStruct((TOKENS * TOPK,), jnp.int32),
            jax.ShapeDtypeStruct((TOKENS * TOPK,), jnp.int32),
            jax.ShapeDtypeStruct((NE,), jnp.int32),
        ],
        mesh=mesh,
        compiler_params=pltpu.CompilerParams(needs_layout_passes=False),
        scratch_types=[
            pltpu.VMEM((16, 128), jnp.int32),
            pltpu.VMEM((16, 128), jnp.int32),
            pltpu.VMEM((NE,), jnp.int32),
            pltpu.VMEM((NE,), jnp.int32),
            pltpu.VMEM((NE,), jnp.int32),
        ] + [pltpu.VMEM((128,), jnp.int32) for _ in range(16)]
          + [pltpu.SemaphoreType.DMA],
    )


def kernel(input, W):
    gate, idx, cumx, tpe, laux = _gate_call(input, W)
    te2d = idx.reshape(NWORK * 16, 128)
    indices, bin_ids, bins = _sort_call()(te2d, tpe, cumx)
    return (laux[0, 0], indices, bin_ids, bins, gate.reshape(-1), tpe[0])


# speedup vs baseline: 1.5380x; 1.5380x over previous
import functools

import jax
import jax.numpy as jnp
from jax import lax
from jax.experimental import pallas as pl
from jax.experimental.pallas import tpu as pltpu
from jax.experimental.pallas import tpu_sc as plsc

TOKENS = 8192
DM = 4096
NE = 64
TOPK = 8
NWORK = 32
CHUNK = TOKENS * TOPK // NWORK
BT = TOKENS // NWORK


def _gate_block(x_ref, w_ref, gate_ref, idx_ref, cumx_ref,
                tpe_ref, laux_ref, me_acc, hist_acc):
    i = pl.program_id(0)

    @pl.when(i == 0)
    def _init():
        me_acc[...] = jnp.zeros_like(me_acc)
        hist_acc[...] = jnp.zeros_like(hist_acc)

    cumx_ref[0, 0, :] = hist_acc[0, :].astype(jnp.int32)

    logits = lax.dot_general(x_ref[...], w_ref[...],
                             (((1,), (1,)), ((), ())),
                             preferred_element_type=jnp.float32)

    iota_f = lax.broadcasted_iota(jnp.int32, (BT, NE), 1).astype(jnp.float32)
    cur = logits
    row_max = None
    gates = []
    idxs = []
    for j in range(TOPK):
        m = jnp.max(cur, axis=1, keepdims=True)
        if j == 0:
            row_max = m
        idx = jnp.min(jnp.where(cur == m, iota_f, float(NE)),
                      axis=1, keepdims=True)
        gates.append(m)
        idxs.append(idx)
        cur = jnp.where(iota_f == idx, -jnp.inf, cur)

    gate_ref[...] = jnp.concatenate(gates, axis=1)
    idx_ref[...] = jnp.concatenate(idxs, axis=1).astype(jnp.int32)

    ex = jnp.exp(logits - row_max)
    scores = ex / jnp.sum(ex, axis=1, keepdims=True)
    me_acc[0, :] = me_acc[0, :] + jnp.sum(scores, axis=0)
    sel_acc = (cur == -jnp.inf).astype(jnp.float32)
    hist_acc[0, :] = hist_acc[0, :] + jnp.sum(sel_acc, axis=0)

    @pl.when(i == NWORK - 1)
    def _final():
        tpe_f = hist_acc[0, :]
        tpe_ref[0, :] = tpe_f.astype(jnp.int32)
        me = me_acc[0, :] * (1.0 / TOKENS)
        ce = tpe_f * (1.0 / TOKENS)
        laux_ref[...] = jnp.sum(me * ce).reshape(1, 1) * (NE / TOPK)


_gate_call = pl.pallas_call(
    _gate_block,
    grid=(NWORK,),
    in_specs=[
        pl.BlockSpec((BT, DM), lambda i: (i, 0)),
        pl.BlockSpec((NE, DM), lambda i: (0, 0)),
    ],
    out_specs=[
        pl.BlockSpec((BT, TOPK), lambda i: (i, 0)),
        pl.BlockSpec((BT, TOPK), lambda i: (i, 0)),
        pl.BlockSpec((1, 1, NE), lambda i: (i, 0, 0)),
        pl.BlockSpec((1, NE), lambda i: (0, 0)),
        pl.BlockSpec((1, 1), lambda i: (0, 0)),
    ],
    out_shape=[
        jax.ShapeDtypeStruct((TOKENS, TOPK), jnp.float32),
        jax.ShapeDtypeStruct((TOKENS, TOPK), jnp.int32),
        jax.ShapeDtypeStruct((NWORK, 1, NE), jnp.int32),
        jax.ShapeDtypeStruct((1, NE), jnp.int32),
        jax.ShapeDtypeStruct((1, 1), jnp.float32),
    ],
    scratch_shapes=[
        pltpu.VMEM((1, NE), jnp.float32),
        pltpu.VMEM((1, NE), jnp.float32),
    ],
)


def _rank_body(te_hbm, tpe_hbm, cumx_hbm, pos_hbm, bins_hbm,
               te_v, pos_v, cnt_v, tmp_v, bin_v):
    c = lax.axis_index("c")
    s = lax.axis_index("s")
    wid = s * 2 + c

    pltpu.sync_copy(te_hbm.at[pl.ds(wid * 16, 16)], te_v)
    pltpu.sync_copy(tpe_hbm.at[0], tmp_v)
    pltpu.sync_copy(cumx_hbm.at[wid, 0], cnt_v)

    carry = jnp.int32(0)
    for t in range(4):
        sl = pl.ds(t * 16, 16)
        v = tmp_v[sl]
        incl = plsc.cumsum(v)
        cnt_v[sl] = cnt_v[sl] + (carry + incl - v)
        bin_v[sl] = carry + incl
        carry = carry + jnp.sum(v)

    @pl.when(wid == 0)
    def _bins_out():
        pltpu.sync_copy(bin_v, bins_hbm)

    for r in range(16):
        for q in range(8):
            sl = pl.ds(q * 16, 16)
            keys = te_v[r, sl]
            cnt, last = plsc.scan_count(keys)
            b = plsc.load_gather(cnt_v, [keys])
            pos_v[r, sl] = b + cnt - 1
            plsc.store_scatter(cnt_v, [keys], b + cnt, mask=last)

    pltpu.sync_copy(pos_v, pos_hbm.at[pl.ds(wid * 16, 16)])


def _permute_body(pos_hbm, bins_hbm, idxout_hbm, binout_hbm,
                  pblk, lin_v, bins_v, idx_loc, bin_loc):
    c = lax.axis_index("c")
    s = lax.axis_index("s")
    wid = s * 2 + c
    lo = wid * CHUNK

    pltpu.sync_copy(bins_hbm, bins_v)
    iota16 = lax.iota(jnp.int32, 16)
    for r in range(32):
        for q in range(8):
            lin_v[r, pl.ds(q * 16, 16)] = r * 128 + q * 16 + iota16

    def blk(b, _):
        pltpu.sync_copy(pos_hbm.at[pl.ds(b * 32, 32)], pblk)
        hi = b * 4096
        for r in range(32):
            for q in range(8):
                sl = pl.ds(q * 16, 16)
                p = pblk[r, sl] - lo
                m = (p >= 0) & (p < CHUNK)
                ps = jnp.where(m, p, 0)
                plsc.store_scatter(idx_loc, [ps], lin_v[r, sl] + hi, mask=m)
        return 0

    lax.fori_loop(0, 16, blk, 0)

    def srch(j, _):
        p = lo + j * 16 + iota16
        e = jnp.zeros((16,), jnp.int32)
        for step in (32, 16, 8, 4, 2, 1):
            cand = e + step
            bv = plsc.load_gather(bins_v, [cand - 1])
            e = jnp.where(bv <= p, cand, e)
        bin_loc[pl.ds(j * 16, 16)] = e
        return 0

    lax.fori_loop(0, CHUNK // 16, srch, 0)

    pltpu.sync_copy(idx_loc, idxout_hbm.at[pl.ds(lo, CHUNK)])
    pltpu.sync_copy(bin_loc, binout_hbm.at[pl.ds(lo, CHUNK)])


@functools.cache
def _sc_calls():
    mesh = plsc.VectorSubcoreMesh(core_axis_name="c", subcore_axis_name="s",
                                  num_cores=2, num_subcores=16)
    cp = pltpu.CompilerParams(needs_layout_passes=False)
    rank_call = pl.kernel(
        _rank_body,
        out_type=[
            jax.ShapeDtypeStruct((512, 128), jnp.int32),
            jax.ShapeDtypeStruct((NE,), jnp.int32),
        ],
        mesh=mesh,
        compiler_params=cp,
        scratch_types=[
            pltpu.VMEM((16, 128), jnp.int32),
            pltpu.VMEM((16, 128), jnp.int32),
            pltpu.VMEM((NE,), jnp.int32),
            pltpu.VMEM((NE,), jnp.int32),
            pltpu.VMEM((NE,), jnp.int32),
        ],
    )
    permute_call = pl.kernel(
        _permute_body,
        out_type=[
            jax.ShapeDtypeStruct((TOKENS * TOPK,), jnp.int32),
            jax.ShapeDtypeStruct((TOKENS * TOPK,), jnp.int32),
        ],
        mesh=mesh,
        compiler_params=cp,
        scratch_types=[
            pltpu.VMEM((32, 128), jnp.int32),
            pltpu.VMEM((32, 128), jnp.int32),
            pltpu.VMEM((NE,), jnp.int32),
            pltpu.VMEM((CHUNK,), jnp.int32),
            pltpu.VMEM((CHUNK,), jnp.int32),
        ],
    )
    return rank_call, permute_call


def kernel(input, W):
    gate, idx, cumx, tpe, laux = _gate_call(input, W)
    rank_call, permute_call = _sc_calls()
    te2d = idx.reshape(NWORK * 16, 128)
    pos, bins = rank_call(te2d, tpe, cumx)
    indices, bin_ids = permute_call(pos, bins)
    return (laux[0, 0], indices, bin_ids, bins, gate.reshape(-1), tpe[0])


# speedup vs baseline: 1.5994x; 1.0399x over previous
import functools

import jax
import jax.numpy as jnp
from jax import lax
from jax.experimental import pallas as pl
from jax.experimental.pallas import tpu as pltpu
from jax.experimental.pallas import tpu_sc as plsc

TOKENS = 8192
DM = 4096
NE = 64
TOPK = 8
NWORK = 32
CHUNK = TOKENS * TOPK // NWORK
BT = TOKENS // NWORK


def _gate_block(x_ref, w_ref, gate_ref, idx_ref, cumx_ref,
                tpe_ref, laux_ref, me_acc, hist_acc):
    i = pl.program_id(0)

    @pl.when(i == 0)
    def _init():
        me_acc[...] = jnp.zeros_like(me_acc)
        hist_acc[...] = jnp.zeros_like(hist_acc)

    cumx_ref[0, 0, :] = hist_acc[0, :].astype(jnp.int32)

    logits = lax.dot_general(x_ref[...], w_ref[...],
                             (((1,), (1,)), ((), ())),
                             preferred_element_type=jnp.float32)

    iota_f = lax.broadcasted_iota(jnp.int32, (BT, NE), 1).astype(jnp.float32)
    cur = logits
    row_max = None
    gates = []
    idxs = []
    for j in range(TOPK):
        m = jnp.max(cur, axis=1, keepdims=True)
        if j == 0:
            row_max = m
        idx = jnp.min(jnp.where(cur == m, iota_f, float(NE)),
                      axis=1, keepdims=True)
        gates.append(m)
        idxs.append(idx)
        cur = jnp.where(iota_f == idx, -jnp.inf, cur)

    gate_ref[...] = jnp.concatenate(gates, axis=1)
    idx_ref[...] = jnp.concatenate(idxs, axis=1).astype(jnp.int32)

    ex = jnp.exp(logits - row_max)
    scores = ex / jnp.sum(ex, axis=1, keepdims=True)
    me_acc[0, :] = me_acc[0, :] + jnp.sum(scores, axis=0)
    sel_acc = (cur == -jnp.inf).astype(jnp.float32)
    hist_acc[0, :] = hist_acc[0, :] + jnp.sum(sel_acc, axis=0)

    @pl.when(i == NWORK - 1)
    def _final():
        tpe_f = hist_acc[0, :]
        tpe_ref[0, :] = tpe_f.astype(jnp.int32)
        me = me_acc[0, :] * (1.0 / TOKENS)
        ce = tpe_f * (1.0 / TOKENS)
        laux_ref[...] = jnp.sum(me * ce).reshape(1, 1) * (NE / TOPK)


_gate_call = pl.pallas_call(
    _gate_block,
    grid=(NWORK,),
    in_specs=[
        pl.BlockSpec((BT, DM), lambda i: (i, 0)),
        pl.BlockSpec((NE, DM), lambda i: (0, 0)),
    ],
    out_specs=[
        pl.BlockSpec((BT, TOPK), lambda i: (i, 0)),
        pl.BlockSpec((BT, TOPK), lambda i: (i, 0)),
        pl.BlockSpec((1, 1, NE), lambda i: (i, 0, 0)),
        pl.BlockSpec((1, NE), lambda i: (0, 0)),
        pl.BlockSpec((1, 1), lambda i: (0, 0)),
    ],
    out_shape=[
        jax.ShapeDtypeStruct((TOKENS, TOPK), jnp.float32),
        jax.ShapeDtypeStruct((TOKENS, TOPK), jnp.int32),
        jax.ShapeDtypeStruct((NWORK, 1, NE), jnp.int32),
        jax.ShapeDtypeStruct((1, NE), jnp.int32),
        jax.ShapeDtypeStruct((1, 1), jnp.float32),
    ],
    scratch_shapes=[
        pltpu.VMEM((1, NE), jnp.float32),
        pltpu.VMEM((1, NE), jnp.float32),
    ],
)


def _rank_body(te_hbm, tpe_hbm, cumx_hbm, pos_hbm, bins_hbm,
               te_v, pos_v, cnt_v, tmp_v, bin_v):
    c = lax.axis_index("c")
    s = lax.axis_index("s")
    wid = s * 2 + c

    pltpu.sync_copy(te_hbm.at[pl.ds(wid * 16, 16)], te_v)
    pltpu.sync_copy(tpe_hbm.at[0], tmp_v)
    pltpu.sync_copy(cumx_hbm.at[wid, 0], cnt_v)

    carry = jnp.int32(0)
    for t in range(4):
        sl = pl.ds(t * 16, 16)
        v = tmp_v[sl]
        incl = plsc.cumsum(v)
        cnt_v[sl] = cnt_v[sl] + (carry + incl - v)
        bin_v[sl] = carry + incl
        carry = carry + jnp.sum(v)

    @pl.when(wid == 0)
    def _bins_out():
        pltpu.sync_copy(bin_v, bins_hbm)

    for r in range(16):
        for q in range(8):
            sl = pl.ds(q * 16, 16)
            keys = te_v[r, sl]
            cnt, last = plsc.scan_count(keys)
            b = plsc.load_gather(cnt_v, [keys])
            pos_v[r, sl] = b + cnt - 1
            plsc.store_scatter(cnt_v, [keys], b + cnt, mask=last)

    pltpu.sync_copy(pos_v, pos_hbm.at[pl.ds(wid * 16, 16)])


def _permute_body(pos_hbm, bins_hbm, idxout_hbm, binout_hbm,
                  pblk0, pblk1, lin_v, bins_v, idx_loc, bin_loc, sem0, sem1):
    c = lax.axis_index("c")
    s = lax.axis_index("s")
    wid = s * 2 + c
    lo = wid * CHUNK

    pltpu.sync_copy(bins_hbm, bins_v)
    iota16 = lax.iota(jnp.int32, 16)
    for r in range(32):
        for q in range(8):
            lin_v[r, pl.ds(q * 16, 16)] = r * 128 + q * 16 + iota16

    def scan_buf(buf, hi):
        for r in range(32):
            for q in range(8):
                sl = pl.ds(q * 16, 16)
                p = buf[r, sl] - lo
                m = (p >= 0) & (p < CHUNK)
                ps = jnp.where(m, p, 0)
                plsc.store_scatter(idx_loc, [ps], lin_v[r, sl] + hi, mask=m)

    pltpu.async_copy(pos_hbm.at[pl.ds(0, 32)], pblk0, sem0)

    def blk(j, _):
        pltpu.make_async_copy(pos_hbm.at[pl.ds(0, 32)], pblk0, sem0).wait()
        pltpu.async_copy(pos_hbm.at[pl.ds((j * 2 + 1) * 32, 32)], pblk1, sem1)
        scan_buf(pblk0, j * 2 * 4096)

        pltpu.make_async_copy(pos_hbm.at[pl.ds(0, 32)], pblk1, sem1).wait()

        @pl.when(j < 7)
        def _pref():
            pltpu.async_copy(pos_hbm.at[pl.ds((j * 2 + 2) * 32, 32)], pblk0, sem0)

        scan_buf(pblk1, (j * 2 + 1) * 4096)
        return 0

    lax.fori_loop(0, 8, blk, 0)

    def srch(j, _):
        p = lo + j * 16 + iota16
        e = jnp.zeros((16,), jnp.int32)
        for step in (32, 16, 8, 4, 2, 1):
            cand = e + step
            bv = plsc.load_gather(bins_v, [cand - 1])
            e = jnp.where(bv <= p, cand, e)
        bin_loc[pl.ds(j * 16, 16)] = e
        return 0

    lax.fori_loop(0, CHUNK // 16, srch, 0)

    pltpu.sync_copy(idx_loc, idxout_hbm.at[pl.ds(lo, CHUNK)])
    pltpu.sync_copy(bin_loc, binout_hbm.at[pl.ds(lo, CHUNK)])


@functools.cache
def _sc_calls():
    mesh = plsc.VectorSubcoreMesh(core_axis_name="c", subcore_axis_name="s",
                                  num_cores=2, num_subcores=16)
    cp = pltpu.CompilerParams(needs_layout_passes=False)
    rank_call = pl.kernel(
        _rank_body,
        out_type=[
            jax.ShapeDtypeStruct((512, 128), jnp.int32),
            jax.ShapeDtypeStruct((NE,), jnp.int32),
        ],
        mesh=mesh,
        compiler_params=cp,
        scratch_types=[
            pltpu.VMEM((16, 128), jnp.int32),
            pltpu.VMEM((16, 128), jnp.int32),
            pltpu.VMEM((NE,), jnp.int32),
            pltpu.VMEM((NE,), jnp.int32),
            pltpu.VMEM((NE,), jnp.int32),
        ],
    )
    permute_call = pl.kernel(
        _permute_body,
        out_type=[
            jax.ShapeDtypeStruct((TOKENS * TOPK,), jnp.int32),
            jax.ShapeDtypeStruct((TOKENS * TOPK,), jnp.int32),
        ],
        mesh=mesh,
        compiler_params=cp,
        scratch_types=[
            pltpu.VMEM((32, 128), jnp.int32),
            pltpu.VMEM((32, 128), jnp.int32),
            pltpu.VMEM((32, 128), jnp.int32),
            pltpu.VMEM((NE,), jnp.int32),
            pltpu.VMEM((CHUNK,), jnp.int32),
            pltpu.VMEM((CHUNK,), jnp.int32),
            pltpu.SemaphoreType.DMA,
            pltpu.SemaphoreType.DMA,
        ],
    )
    return rank_call, permute_call


def kernel(input, W):
    gate, idx, cumx, tpe, laux = _gate_call(input, W)
    rank_call, permute_call = _sc_calls()
    te2d = idx.reshape(NWORK * 16, 128)
    pos, bins = rank_call(te2d, tpe, cumx)
    indices, bin_ids = permute_call(pos, bins)
    return (laux[0, 0], indices, bin_ids, bins, gate.reshape(-1), tpe[0])
